# baseline (speedup 1.0000x reference)
import functools

import jax
import jax.numpy as jnp
from jax import lax
from jax.experimental import pallas as pl
from jax.experimental.pallas import tpu as pltpu
from jax.experimental.pallas import tpu_sc as plsc

D_MODEL = 64

_NC = 2
_NS = 16
_NW = _NC * _NS

_GROUP = 128
_JPG = 4
_CHUNK = _GROUP * _JPG


def _make_kernel(n_rows: int):
    rows_per_w = n_rows // _NW
    n_chunks = rows_per_w // _CHUNK
    mesh = plsc.VectorSubcoreMesh(core_axis_name="c", subcore_axis_name="s")

    @functools.partial(
        pl.kernel,
        mesh=mesh,
        out_type=jax.ShapeDtypeStruct((n_rows, D_MODEL), jnp.float32),
        compiler_params=pltpu.CompilerParams(use_tc_tiling_on_sc=False),
        scratch_types=[
            pltpu.VMEM((_JPG, _GROUP), jnp.int32),
            pltpu.VMEM((_CHUNK, D_MODEL), jnp.float32),
            pltpu.SemaphoreType.DMA,
        ],
    )
    def k(table_hbm, idx_hbm, out_hbm, idx_v, rows_v, sem):
        wid = lax.axis_index("s") * _NC + lax.axis_index("c")
        base = wid * rows_per_w

        def body(c, carry):
            pltpu.sync_copy(idx_hbm.at[wid, c], idx_v)
            copies = []
            for j in range(_JPG):
                copies.append(
                    pltpu.async_copy(
                        table_hbm.at[idx_v.at[j]],
                        rows_v.at[pl.ds(j * _GROUP, _GROUP), :],
                        sem,
                    )
                )
            for cp in copies:
                cp.wait()
            pltpu.sync_copy(
                rows_v, out_hbm.at[pl.ds(base + c * _CHUNK, _CHUNK), :]
            )
            return carry

        lax.fori_loop(0, n_chunks, body, 0)

    return k


def kernel(x, table):
    b, s = x.shape
    n_rows = b * s
    idx = x.reshape(_NW, (n_rows // _NW) // _CHUNK, _JPG, _GROUP).astype(jnp.int32)
    out = _make_kernel(n_rows)(table, idx)
    return out.reshape(b, s, D_MODEL)


# speedup vs baseline: 1.0202x; 1.0202x over previous
import functools

import jax
import jax.numpy as jnp
from jax import lax
from jax.experimental import pallas as pl
from jax.experimental.pallas import tpu as pltpu
from jax.experimental.pallas import tpu_sc as plsc

D_MODEL = 64

_NC = 2
_NS = 16
_NW = _NC * _NS

_GROUP = 128
_JPG = 2
_CHUNK = _GROUP * _JPG
_NBUF = 5


def _make_kernel(n_rows: int):
    rows_per_w = n_rows // _NW
    groups_per_w = rows_per_w // _GROUP
    n_chunks = rows_per_w // _CHUNK
    n_steps = n_chunks // _NBUF
    chunk_bytes_shape = jax.ShapeDtypeStruct((_CHUNK, D_MODEL), jnp.float32)
    del chunk_bytes_shape
    mesh = plsc.VectorSubcoreMesh(core_axis_name="c", subcore_axis_name="s")

    @functools.partial(
        pl.kernel,
        mesh=mesh,
        out_type=jax.ShapeDtypeStruct((n_rows, D_MODEL), jnp.float32),
        compiler_params=pltpu.CompilerParams(use_tc_tiling_on_sc=False),
        scratch_types=(
            [pltpu.VMEM((groups_per_w, _GROUP), jnp.int32)]
            + [pltpu.VMEM((_CHUNK, D_MODEL), jnp.float32) for _ in range(_NBUF)]
            + [pltpu.SemaphoreType.DMA for _ in range(2 * _NBUF)]
        ),
    )
    def k(table_hbm, idx_hbm, out_hbm, idx_all, *rest):
        rows = rest[:_NBUF]
        gsem = rest[_NBUF : 2 * _NBUF]
        wsem = rest[2 * _NBUF :]
        wid = lax.axis_index("s") * _NC + lax.axis_index("c")
        base = wid * rows_per_w

        pltpu.sync_copy(idx_hbm.at[wid], idx_all)

        def drain(sem, b):
            pltpu.make_async_copy(
                out_hbm.at[pl.ds(0, _CHUNK), :], rows[b], sem
            ).wait()

        def body(s, carry):
            for b in range(_NBUF):
                c = s * _NBUF + b

                @pl.when(s > 0)
                def _():
                    drain(wsem[b], b)

                for j in range(_JPG):
                    g = c * _JPG + j
                    pltpu.async_copy(
                        table_hbm.at[idx_all.at[g]],
                        rows[b].at[pl.ds(j * _GROUP, _GROUP), :],
                        gsem[b],
                    )
            for b in range(_NBUF):
                c = s * _NBUF + b
                drain(gsem[b], b)
                pltpu.async_copy(
                    rows[b],
                    out_hbm.at[pl.ds(base + c * _CHUNK, _CHUNK), :],
                    wsem[b],
                )
            return carry

        lax.fori_loop(0, n_steps, body, 0)
        for b in range(_NBUF):
            drain(wsem[b], b)

    return k


def kernel(x, table):
    b, s = x.shape
    n_rows = b * s
    idx = x.reshape(_NW, (n_rows // _NW) // _GROUP, _GROUP).astype(jnp.int32)
    out = _make_kernel(n_rows)(table, idx)
    return out.reshape(b, s, D_MODEL)
